# baseline (speedup 1.0000x reference)
import functools

import jax
import jax.numpy as jnp
from jax import lax
from jax.experimental import pallas as pl
from jax.experimental.pallas import tpu as pltpu
from jax.experimental.pallas import tpu_sc as plsc

_PIECE_I = 4
_NBUF = 2


def kernel(context_mapping, table):
    n, l = context_mapping.shape
    d = table.shape[1]
    l2 = l // 2
    b2 = n * l2

    info = plsc.get_sparse_core_info()
    nw = info.num_cores * info.num_subcores
    lanes = info.num_lanes

    rows_pw = n // nw
    piece = _PIECE_I * l2
    n_pieces = rows_pw // _PIECE_I
    n_outer = n_pieces // _NBUF
    n_mul = piece // lanes
    spans = []
    s = 0
    while s < piece:
        ln = min(128, piece - s)
        spans.append((s, ln))
        s += ln

    t_even = table[0:l:2, :]
    t_odd = table[1:l:2, :]
    t_zero = jnp.broadcast_to(table[0:1, :], (l2, d))
    ptable = jnp.concatenate([
        jnp.concatenate([t_zero, t_zero], axis=1),
        jnp.concatenate([t_even, t_zero], axis=1),
        jnp.concatenate([t_zero, t_odd], axis=1),
        jnp.concatenate([t_even, t_odd], axis=1),
    ], axis=0)

    me_flat = context_mapping[:, 0::2].reshape(b2).astype(jnp.int32)
    mo_flat = context_mapping[:, 1::2].reshape(b2).astype(jnp.int32)
    jseq = jnp.tile(jnp.arange(l2, dtype=jnp.int32), _PIECE_I)

    scratch = [pltpu.VMEM((piece,), jnp.int32)]
    for _ in range(_NBUF):
        scratch += [pltpu.VMEM((piece,), jnp.int32),
                    pltpu.VMEM((piece,), jnp.int32),
                    pltpu.VMEM((piece,), jnp.int32),
                    pltpu.VMEM((piece, 2 * d), jnp.float32),
                    pltpu.SemaphoreType.DMA,
                    pltpu.SemaphoreType.DMA,
                    pltpu.SemaphoreType.DMA]

    @functools.partial(
        pl.kernel,
        mesh=plsc.VectorSubcoreMesh(core_axis_name="c", subcore_axis_name="s"),
        out_type=jax.ShapeDtypeStruct((b2, 2 * d), jnp.float32),
        scratch_types=scratch,
    )
    def sc_gather(me_hbm, mo_hbm, jseq_hbm, ptable_hbm, out_hbm,
                  jseq_v, *bufs):
        wid = lax.axis_index("s") * info.num_cores + lax.axis_index("c")
        base = wid * rows_pw * l2
        rings = [bufs[7 * i:7 * i + 7] for i in range(_NBUF)]
        pltpu.sync_copy(jseq_hbm, jseq_v)

        def fire_m(t, r):
            off = pl.multiple_of(base + t * piece, 8)
            pltpu.async_copy(me_hbm.at[pl.ds(off, piece)], r[0], r[4])
            pltpu.async_copy(mo_hbm.at[pl.ds(off, piece)], r[1], r[4])

        def drain_m(r):
            pltpu.make_async_copy(
                me_hbm.at[pl.ds(0, piece)], r[0], r[4]).wait()
            pltpu.make_async_copy(
                mo_hbm.at[pl.ds(0, piece)], r[1], r[4]).wait()

        def fire_g(r):
            for (s0, ln) in spans:
                pltpu.async_copy(
                    ptable_hbm.at[r[2].at[pl.ds(s0, ln)]],
                    r[3].at[pl.ds(s0, ln)], r[5])

        def drain_g(r):
            for (s0, ln) in spans:
                pltpu.make_async_copy(
                    ptable_hbm.at[r[2].at[pl.ds(s0, ln)]],
                    r[3].at[pl.ds(s0, ln)], r[5]).wait()

        def fire_o(t, r):
            off = pl.multiple_of(base + t * piece, 8)
            pltpu.async_copy(r[3], out_hbm.at[pl.ds(off, piece)], r[6])

        def drain_o(r):
            pltpu.make_async_copy(
                r[3], out_hbm.at[pl.ds(base, piece)], r[6]).wait()

        for bi in range(_NBUF):
            fire_m(bi, rings[bi])

        def compute_idx(r):
            for v in range(n_mul):
                sl = pl.ds(v * lanes, lanes)
                r[2][sl] = (r[0][sl] + 2 * r[1][sl]) * l2 + jseq_v[sl]

        def body(g, carry):
            for bi in range(_NBUF):
                t = _NBUF * g + bi
                r = rings[bi]
                rprev = rings[(bi - 1) % _NBUF]
                drain_m(r)

                @pl.when(g >= 1)
                def _():
                    drain_o(r)

                compute_idx(r)
                fire_g(r)

                if bi >= 1:
                    drain_g(rprev)
                    fire_o(t - 1, rprev)
                else:
                    @pl.when(g >= 1)
                    def _():
                        drain_g(rprev)
                        fire_o(t - 1, rprev)

                @pl.when(g <= n_outer - 2)
                def _():
                    fire_m(t + _NBUF, r)
            return carry

        lax.fori_loop(0, n_outer, body, 0)

        last = rings[(n_pieces - 1) % _NBUF]
        drain_g(last)
        fire_o(n_pieces - 1, last)
        for bi in range(_NBUF):
            drain_o(rings[bi])

    out = sc_gather(me_flat, mo_flat, jseq, ptable)
    return out.reshape(n, l, d)
